# baseline (speedup 1.0000x reference)
import functools

import jax
import jax.numpy as jnp
from jax import lax
from jax.experimental import pallas as pl
from jax.experimental.pallas import tpu as pltpu
from jax.experimental.pallas import tpu_sc as plsc

D_MODEL = 768
D_HID = 256
D_FF = 2 * D_HID
N_EXP = 64
T = 2048

NW = 32
BPW = T // NW
NWR = 16
TPW = T // NWR
TM = 256

_IB = lax.GatherScatterMode.PROMISE_IN_BOUNDS


def _take(v, idx):
    return v.at[idx].get(mode="promise_in_bounds")


def _cumsum16(v, lane):
    zero = jnp.zeros((16,), jnp.int32)
    for sh in (1, 2, 4, 8):
        shifted = _take(v, jnp.maximum(lane - sh, 0))
        v = v + jnp.where(lane >= sh, shifted, zero)
    return v


def _sc_route(e_ids):
    mesh = plsc.VectorSubcoreMesh(core_axis_name="c", subcore_axis_name="s")

    @functools.partial(
        pl.kernel,
        mesh=mesh,
        out_type=[
            jax.ShapeDtypeStruct((T,), jnp.int32),
            jax.ShapeDtypeStruct((80,), jnp.int32),
        ],
        scratch_types=[
            pltpu.VMEM((TPW,), jnp.int32),
            pltpu.VMEM((TPW,), jnp.int32),
            pltpu.VMEM((N_EXP,), jnp.int32),
            pltpu.VMEM((NWR * N_EXP,), jnp.int32),
            pltpu.VMEM((80,), jnp.int32),
            pltpu.VMEM_SHARED((NWR * N_EXP,), jnp.int32),
        ],
    )
    def route_k(e_hbm, rank_hbm, offs_hbm,
                ids_v, pos_v, hist_v, grid_v, offs_v, shared):
        cid = lax.axis_index("c")
        sid = lax.axis_index("s")

        @pl.when(cid == 0)
        def _body():
            lane = lax.broadcasted_iota(jnp.int32, (16,), 0)
            zero = jnp.zeros((16,), jnp.int32)
            ev = [lane + 16 * g for g in range(4)]
            base_t = sid * TPW
            pltpu.sync_copy(e_hbm.at[pl.ds(base_t, TPW)], ids_v)

            def outer(r, bs):
                idv = ids_v[pl.ds(r * 16, 16)]
                posreg = zero
                b = list(bs)
                for k in range(16):
                    idb = _take(idv, jnp.full((16,), k, jnp.int32))
                    lsel = idb & 15
                    grp = idb >> 4
                    pos = zero
                    for g in range(4):
                        pos = pos + jnp.where(grp == g,
                                              _take(b[g], lsel), zero)
                    posreg = posreg + jnp.where(lane == k, pos, zero)
                    for g in range(4):
                        b[g] = b[g] + jnp.where(ev[g] == idb, 1, 0)
                pos_v[pl.ds(r * 16, 16)] = posreg
                return tuple(b)

            bfin = lax.fori_loop(0, TPW // 16, outer,
                                 (zero, zero, zero, zero))
            for g in range(4):
                hist_v[pl.ds(g * 16, 16)] = bfin[g]
            pltpu.sync_copy(hist_v, shared.at[pl.ds(sid * N_EXP, N_EXP)])
            plsc.subcore_barrier()
            pltpu.sync_copy(shared, grid_v)

            sid_v = jnp.broadcast_to(sid, (16,))
            c = [zero] * 4
            pw = [zero] * 4
            for w in range(NWR):
                wv = jnp.full((16,), w, jnp.int32)
                for g in range(4):
                    row = grid_v[pl.ds(w * N_EXP + g * 16, 16)]
                    c[g] = c[g] + row
                    wlt = jnp.minimum(jnp.maximum(sid_v - wv, 0), 1)
                    pw[g] = pw[g] + row * wlt
            cs = [_cumsum16(c[g], lane) for g in range(4)]
            excl = [cs[g] - c[g] for g in range(4)]
            tot = [_take(cs[g], jnp.full((16,), 15, jnp.int32))
                   for g in range(4)]
            basev = [excl[0]]
            run = tot[0]
            for g in range(1, 4):
                basev.append(excl[g] + run)
                run = run + tot[g]
            wb = [basev[g] + pw[g] for g in range(4)]

            @pl.when(sid == 0)
            def _offs():
                for g in range(4):
                    offs_v[pl.ds(g * 16, 16)] = basev[g]
                offs_v[pl.ds(64, 16)] = jnp.full((16,), T, jnp.int32)
                pltpu.sync_copy(offs_v, offs_hbm)

            def fix(r, carry):
                idv = ids_v[pl.ds(r * 16, 16)]
                lp = pos_v[pl.ds(r * 16, 16)]
                lsel = idv & 15
                grp = idv >> 4
                add = zero
                for g in range(4):
                    add = add + jnp.where(grp == g, _take(wb[g], lsel), zero)
                pos_v[pl.ds(r * 16, 16)] = lp + add
                return carry

            lax.fori_loop(0, TPW // 16, fix, 0)
            pltpu.sync_copy(pos_v, rank_hbm.at[pl.ds(base_t, TPW)])

    return route_k(e_ids)


def _sc_dispatch(x, scale2d, rank):
    mesh = plsc.VectorSubcoreMesh(core_axis_name="c", subcore_axis_name="s")

    @functools.partial(
        pl.kernel,
        mesh=mesh,
        out_type=[
            jax.ShapeDtypeStruct((T, D_MODEL), jnp.float32),
            jax.ShapeDtypeStruct((T, 128), jnp.float32),
        ],
        scratch_types=[
            pltpu.VMEM((BPW,), jnp.int32),
            pltpu.VMEM((BPW, D_MODEL), jnp.float32),
            pltpu.VMEM((BPW, 128), jnp.float32),
            pltpu.SemaphoreType.DMA,
            pltpu.SemaphoreType.DMA,
        ],
    )
    def dispatch_k(x_hbm, scale_hbm, rank_hbm, xs_hbm, ss_hbm,
                   idx_v, rows_v, srows_v, sem1, sem2):
        wid = lax.axis_index("s") * 2 + lax.axis_index("c")
        base = wid * BPW
        pltpu.sync_copy(rank_hbm.at[pl.ds(base, BPW)], idx_v)
        pltpu.sync_copy(x_hbm.at[pl.ds(base, BPW)], rows_v)
        pltpu.sync_copy(scale_hbm.at[pl.ds(base, BPW)], srows_v)
        cp1 = pltpu.async_copy(rows_v, xs_hbm.at[idx_v], sem1)
        cp2 = pltpu.async_copy(srows_v, ss_hbm.at[idx_v], sem2)
        cp1.wait()
        cp2.wait()

    return dispatch_k(x, scale2d, rank)


def _sc_combine(table, rank):
    mesh = plsc.VectorSubcoreMesh(core_axis_name="c", subcore_axis_name="s")

    @functools.partial(
        pl.kernel,
        mesh=mesh,
        out_type=jax.ShapeDtypeStruct((T, D_MODEL), jnp.float32),
        scratch_types=[
            pltpu.VMEM((BPW,), jnp.int32),
            pltpu.VMEM((BPW, D_MODEL), jnp.float32),
            pltpu.SemaphoreType.DMA,
        ],
    )
    def combine_k(table_hbm, idx_hbm, out_hbm, idx_v, rows_v, sem):
        wid = lax.axis_index("s") * 2 + lax.axis_index("c")
        base = wid * BPW
        pltpu.sync_copy(idx_hbm.at[pl.ds(base, BPW)], idx_v)
        pltpu.async_copy(table_hbm.at[idx_v], rows_v, sem).wait()
        pltpu.sync_copy(rows_v, out_hbm.at[pl.ds(base, BPW)])

    return combine_k(table, rank)


def _tc_gmm_kernel(offs_ref, xs_ref, ss_ref, w1_ref, w2_ref, out_ref):
    e = pl.program_id(0)

    @pl.when(e == 0)
    def _init():
        out_ref[...] = jnp.zeros_like(out_ref)

    start = offs_ref[e]
    end = offs_ref[e + 1]
    t0 = start // TM
    t1 = (end + TM - 1) // TM
    w1 = w1_ref[0]
    w2 = w2_ref[0]

    def body(ti, carry):
        r0 = ti * TM
        rows = xs_ref[pl.ds(r0, TM), :]
        y = lax.dot_general(rows, w1, (((1,), (1,)), ((), ())),
                            preferred_element_type=jnp.float32)
        y1 = y[:, :D_HID]
        g = y[:, D_HID:]
        rid = r0 + lax.broadcasted_iota(jnp.int32, (TM, 1), 0)
        m = (rid >= start) & (rid < end)
        sc = ss_ref[pl.ds(r0, TM), 0:1]
        msc = jnp.where(m, sc, 0.0)
        h = y1 * g * jax.nn.sigmoid(g) * msc
        yo = lax.dot_general(h, w2, (((1,), (1,)), ((), ())),
                             preferred_element_type=jnp.float32)
        out_ref[pl.ds(r0, TM), :] += yo
        return carry

    lax.fori_loop(t0, t1, body, 0)


def _tc_gmm(offsets, xs, ss, fc1_weights, fc2_weights):
    return pl.pallas_call(
        _tc_gmm_kernel,
        grid=(N_EXP,),
        in_specs=[
            pl.BlockSpec(memory_space=pltpu.SMEM),
            pl.BlockSpec((T, D_MODEL), lambda e: (0, 0)),
            pl.BlockSpec((T, 128), lambda e: (0, 0)),
            pl.BlockSpec((1, D_FF, D_MODEL), lambda e: (e, 0, 0)),
            pl.BlockSpec((1, D_MODEL, D_HID), lambda e: (e, 0, 0)),
        ],
        out_specs=pl.BlockSpec((T, D_MODEL), lambda e: (0, 0)),
        out_shape=jax.ShapeDtypeStruct((T, D_MODEL), jnp.float32),
        compiler_params=pltpu.CompilerParams(
            dimension_semantics=("arbitrary",)),
    )(offsets, xs, ss, fc1_weights, fc2_weights)


def kernel(x, weights, indices, fc1_weights, fc2_weights):
    e_ids = indices[:, 0].astype(jnp.int32)
    scale = weights[:, 0].astype(jnp.float32)
    scale2d = jnp.broadcast_to(scale[:, None], (T, 128))

    rank, offsets = _sc_route(e_ids)
    xs, ss = _sc_dispatch(x, scale2d, rank)
    out_sorted = _tc_gmm(offsets, xs, ss, fc1_weights, fc2_weights)
    return _sc_combine(out_sorted, rank)


# speedup vs baseline: 1.0317x; 1.0317x over previous
import functools

import jax
import jax.numpy as jnp
from jax import lax
from jax.experimental import pallas as pl
from jax.experimental.pallas import tpu as pltpu
from jax.experimental.pallas import tpu_sc as plsc

D_MODEL = 768
D_HID = 256
D_FF = 2 * D_HID
N_EXP = 64
T = 2048

NW = 32
BPW = T // NW
NWR = 16
TPW = T // NWR
TM = 256

_IB = lax.GatherScatterMode.PROMISE_IN_BOUNDS


def _take(v, idx):
    return v.at[idx].get(mode="promise_in_bounds")


def _cumsum16(v, lane):
    zero = jnp.zeros((16,), jnp.int32)
    for sh in (1, 2, 4, 8):
        shifted = _take(v, jnp.maximum(lane - sh, 0))
        v = v + jnp.where(lane >= sh, shifted, zero)
    return v


def _sc_route_dispatch(e_ids, x, scale2d):
    mesh = plsc.VectorSubcoreMesh(core_axis_name="c", subcore_axis_name="s")

    @functools.partial(
        pl.kernel,
        mesh=mesh,
        out_type=[
            jax.ShapeDtypeStruct((T,), jnp.int32),
            jax.ShapeDtypeStruct((80,), jnp.int32),
            jax.ShapeDtypeStruct((T, D_MODEL), jnp.float32),
            jax.ShapeDtypeStruct((T, 128), jnp.float32),
        ],
        scratch_types=[
            pltpu.VMEM((TPW,), jnp.int32),
            pltpu.VMEM((TPW,), jnp.int32),
            pltpu.VMEM((N_EXP,), jnp.int32),
            pltpu.VMEM((NWR * N_EXP,), jnp.int32),
            pltpu.VMEM((80,), jnp.int32),
            pltpu.VMEM((BPW,), jnp.int32),
            pltpu.VMEM((BPW, D_MODEL), jnp.float32),
            pltpu.VMEM((BPW, 128), jnp.float32),
            pltpu.VMEM_SHARED((NWR * N_EXP,), jnp.int32),
            pltpu.SemaphoreType.DMA,
            pltpu.SemaphoreType.DMA,
        ],
    )
    def route_k(e_hbm, x_hbm, scale_hbm, rank_hbm, offs_hbm, xs_hbm, ss_hbm,
                ids_v, pos_v, hist_v, grid_v, offs_v,
                idx_v, rows_v, srows_v, shared, sem1, sem2):
        cid = lax.axis_index("c")
        sid = lax.axis_index("s")
        if True:
            lane = lax.broadcasted_iota(jnp.int32, (16,), 0)
            zero = jnp.zeros((16,), jnp.int32)
            ev = [lane + 16 * g for g in range(4)]
            base_t = sid * TPW
            pltpu.sync_copy(e_hbm.at[pl.ds(base_t, TPW)], ids_v)

            def outer(r, bs):
                idv = ids_v[pl.ds(r * 16, 16)]
                posreg = zero
                b = list(bs)
                for k in range(16):
                    idb = _take(idv, jnp.full((16,), k, jnp.int32))
                    lsel = idb & 15
                    grp = idb >> 4
                    pos = zero
                    for g in range(4):
                        pos = pos + jnp.where(grp == g,
                                              _take(b[g], lsel), zero)
                    posreg = posreg + jnp.where(lane == k, pos, zero)
                    for g in range(4):
                        b[g] = b[g] + jnp.where(ev[g] == idb, 1, 0)
                pos_v[pl.ds(r * 16, 16)] = posreg
                return tuple(b)

            bfin = lax.fori_loop(0, TPW // 16, outer,
                                 (zero, zero, zero, zero))
            for g in range(4):
                hist_v[pl.ds(g * 16, 16)] = bfin[g]
            pltpu.sync_copy(hist_v, shared.at[pl.ds(sid * N_EXP, N_EXP)])
            plsc.subcore_barrier()
            pltpu.sync_copy(shared, grid_v)

            sid_v = jnp.broadcast_to(sid, (16,))
            c = [zero] * 4
            pw = [zero] * 4
            for w in range(NWR):
                wv = jnp.full((16,), w, jnp.int32)
                for g in range(4):
                    row = grid_v[pl.ds(w * N_EXP + g * 16, 16)]
                    c[g] = c[g] + row
                    wlt = jnp.minimum(jnp.maximum(sid_v - wv, 0), 1)
                    pw[g] = pw[g] + row * wlt
            cs = [_cumsum16(c[g], lane) for g in range(4)]
            excl = [cs[g] - c[g] for g in range(4)]
            tot = [_take(cs[g], jnp.full((16,), 15, jnp.int32))
                   for g in range(4)]
            basev = [excl[0]]
            run = tot[0]
            for g in range(1, 4):
                basev.append(excl[g] + run)
                run = run + tot[g]
            wb = [basev[g] + pw[g] for g in range(4)]

            @pl.when((sid == 0) & (cid == 0))
            def _offs():
                for g in range(4):
                    offs_v[pl.ds(g * 16, 16)] = basev[g]
                offs_v[pl.ds(64, 16)] = jnp.full((16,), T, jnp.int32)
                pltpu.sync_copy(offs_v, offs_hbm)

            def fix(r, carry):
                idv = ids_v[pl.ds(r * 16, 16)]
                lp = pos_v[pl.ds(r * 16, 16)]
                lsel = idv & 15
                grp = idv >> 4
                add = zero
                for g in range(4):
                    add = add + jnp.where(grp == g, _take(wb[g], lsel), zero)
                pos_v[pl.ds(r * 16, 16)] = lp + add
                return carry

            lax.fori_loop(0, TPW // 16, fix, 0)

            half = cid * BPW
            for q in range(BPW // 16):
                idx_v[pl.ds(q * 16, 16)] = pos_v[pl.ds(half + q * 16, 16)]
            dbase = base_t + half
            pltpu.sync_copy(idx_v, rank_hbm.at[pl.ds(dbase, BPW)])
            pltpu.sync_copy(x_hbm.at[pl.ds(dbase, BPW)], rows_v)
            pltpu.sync_copy(scale_hbm.at[pl.ds(dbase, BPW)], srows_v)
            cp1 = pltpu.async_copy(rows_v, xs_hbm.at[idx_v], sem1)
            cp2 = pltpu.async_copy(srows_v, ss_hbm.at[idx_v], sem2)
            cp1.wait()
            cp2.wait()

    return route_k(e_ids, x, scale2d)


def _sc_dispatch(x, scale2d, rank):
    mesh = plsc.VectorSubcoreMesh(core_axis_name="c", subcore_axis_name="s")

    @functools.partial(
        pl.kernel,
        mesh=mesh,
        out_type=[
            jax.ShapeDtypeStruct((T, D_MODEL), jnp.float32),
            jax.ShapeDtypeStruct((T, 128), jnp.float32),
        ],
        scratch_types=[
            pltpu.VMEM((BPW,), jnp.int32),
            pltpu.VMEM((BPW, D_MODEL), jnp.float32),
            pltpu.VMEM((BPW, 128), jnp.float32),
            pltpu.SemaphoreType.DMA,
            pltpu.SemaphoreType.DMA,
        ],
    )
    def dispatch_k(x_hbm, scale_hbm, rank_hbm, xs_hbm, ss_hbm,
                   idx_v, rows_v, srows_v, sem1, sem2):
        wid = lax.axis_index("s") * 2 + lax.axis_index("c")
        base = wid * BPW
        pltpu.sync_copy(rank_hbm.at[pl.ds(base, BPW)], idx_v)
        pltpu.sync_copy(x_hbm.at[pl.ds(base, BPW)], rows_v)
        pltpu.sync_copy(scale_hbm.at[pl.ds(base, BPW)], srows_v)
        cp1 = pltpu.async_copy(rows_v, xs_hbm.at[idx_v], sem1)
        cp2 = pltpu.async_copy(srows_v, ss_hbm.at[idx_v], sem2)
        cp1.wait()
        cp2.wait()

    return dispatch_k(x, scale2d, rank)


def _sc_combine(table, rank):
    mesh = plsc.VectorSubcoreMesh(core_axis_name="c", subcore_axis_name="s")

    @functools.partial(
        pl.kernel,
        mesh=mesh,
        out_type=jax.ShapeDtypeStruct((T, D_MODEL), jnp.float32),
        scratch_types=[
            pltpu.VMEM((BPW,), jnp.int32),
            pltpu.VMEM((BPW, D_MODEL), jnp.float32),
            pltpu.SemaphoreType.DMA,
        ],
    )
    def combine_k(table_hbm, idx_hbm, out_hbm, idx_v, rows_v, sem):
        wid = lax.axis_index("s") * 2 + lax.axis_index("c")
        base = wid * BPW
        pltpu.sync_copy(idx_hbm.at[pl.ds(base, BPW)], idx_v)
        pltpu.async_copy(table_hbm.at[idx_v], rows_v, sem).wait()
        pltpu.sync_copy(rows_v, out_hbm.at[pl.ds(base, BPW)])

    return combine_k(table, rank)


def _tc_gmm_kernel(offs_ref, xs_ref, ss_ref, w1_ref, w2_ref, out_ref):
    e = pl.program_id(0)

    @pl.when(e == 0)
    def _init():
        out_ref[...] = jnp.zeros_like(out_ref)

    start = offs_ref[e]
    end = offs_ref[e + 1]
    t0 = start // TM
    t1 = (end + TM - 1) // TM
    w1 = w1_ref[0]
    w2 = w2_ref[0]

    def body(ti, carry):
        r0 = ti * TM
        rows = xs_ref[pl.ds(r0, TM), :]
        y = lax.dot_general(rows, w1, (((1,), (1,)), ((), ())),
                            preferred_element_type=jnp.float32)
        y1 = y[:, :D_HID]
        g = y[:, D_HID:]
        rid = r0 + lax.broadcasted_iota(jnp.int32, (TM, 1), 0)
        m = (rid >= start) & (rid < end)
        sc = ss_ref[pl.ds(r0, TM), 0:1]
        msc = jnp.where(m, sc, 0.0)
        h = y1 * g * jax.nn.sigmoid(g) * msc
        yo = lax.dot_general(h, w2, (((1,), (1,)), ((), ())),
                             preferred_element_type=jnp.float32)
        out_ref[pl.ds(r0, TM), :] += yo
        return carry

    lax.fori_loop(t0, t1, body, 0)


def _tc_gmm(offsets, xs, ss, fc1_weights, fc2_weights):
    return pl.pallas_call(
        _tc_gmm_kernel,
        grid=(N_EXP,),
        in_specs=[
            pl.BlockSpec(memory_space=pltpu.SMEM),
            pl.BlockSpec((T, D_MODEL), lambda e: (0, 0)),
            pl.BlockSpec((T, 128), lambda e: (0, 0)),
            pl.BlockSpec((1, D_FF, D_MODEL), lambda e: (e, 0, 0)),
            pl.BlockSpec((1, D_MODEL, D_HID), lambda e: (e, 0, 0)),
        ],
        out_specs=pl.BlockSpec((T, D_MODEL), lambda e: (0, 0)),
        out_shape=jax.ShapeDtypeStruct((T, D_MODEL), jnp.float32),
        compiler_params=pltpu.CompilerParams(
            dimension_semantics=("arbitrary",)),
    )(offsets, xs, ss, fc1_weights, fc2_weights)


def kernel(x, weights, indices, fc1_weights, fc2_weights):
    e_ids = indices[:, 0].astype(jnp.int32)
    scale = weights[:, 0].astype(jnp.float32)
    scale2d = jnp.broadcast_to(scale[:, None], (T, 128))

    rank, offsets, xs, ss = _sc_route_dispatch(e_ids, x, scale2d)
    out_sorted = _tc_gmm(offsets, xs, ss, fc1_weights, fc2_weights)
    return _sc_combine(out_sorted, rank)
